# baseline (speedup 1.0000x reference)
import functools

import jax
import jax.numpy as jnp
from jax import lax
from jax.experimental import pallas as pl
from jax.experimental.pallas import tpu as pltpu
from jax.experimental.pallas import tpu_sc as plsc

N = 10000
K = 16
R = 200
S = 128
C = 79
NB = C * S
CW = K * S

NW = 32
RW = 320
NP = NW * RW
EW = RW * K
EP = NP * K


def _batcher_pairs(n):
    pairs = []
    p = 1
    while p < n:
        k = p
        while k >= 1:
            for j in range(k % p, n - k, 2 * k):
                for i in range(min(k, n - j - k)):
                    if (i + j) // (2 * p) == (i + j + k) // (2 * p):
                        pairs.append((i + j, i + j + k))
            k //= 2
        p *= 2
    return pairs


_SORT16 = _batcher_pairs(K)
_SORT17 = _batcher_pairs(K + 1)


def _tc_a_body(dst_ref, srcT_ref, d2_ref, csel_ref):
    dblk = dst_ref[...]
    sT = srcT_ref[...]
    dn2 = jnp.sum(dblk * dblk, axis=1, keepdims=True)
    sn2 = jnp.sum(sT * sT, axis=0, keepdims=True)
    cross = jnp.dot(dblk, sT, preferred_element_type=jnp.float32)
    d2 = (dn2 - 2.0 * cross) + sn2
    inf = jnp.float32(jnp.inf)
    d2p = jnp.concatenate(
        [d2, jnp.full((R, NB - N), inf, jnp.float32)], axis=1)
    d2_ref[...] = d2p
    mins = [jnp.min(d2p[:, c * S:(c + 1) * S], axis=1, keepdims=True)
            for c in range(C)]
    M = jnp.concatenate(
        mins + [jnp.full((R, 128 - C), inf, jnp.float32)], axis=1)
    iota = lax.broadcasted_iota(jnp.int32, (R, 128), 1)
    bigi = jnp.int32(2**30)
    cols = []
    for _ in range(K):
        m = jnp.min(M, axis=1, keepdims=True)
        sel = jnp.min(jnp.where(M == m, iota, bigi), axis=1, keepdims=True)
        M = jnp.where(iota == sel, inf, M)
        cols.append(sel)
    for a, b in _SORT16:
        lo = jnp.minimum(cols[a], cols[b])
        hi = jnp.maximum(cols[a], cols[b])
        cols[a], cols[b] = lo, hi
    csel_ref[...] = jnp.concatenate(cols, axis=1)


@jax.jit
def _tc_a(dst_pad, srcT_pad):
    return pl.pallas_call(
        _tc_a_body,
        grid=(N // R,),
        in_specs=[
            pl.BlockSpec((R, 8), lambda i: (i, 0)),
            pl.BlockSpec((8, N), lambda i: (0, 0)),
        ],
        out_specs=[
            pl.BlockSpec((R, NB), lambda i: (i, 0)),
            pl.BlockSpec((R, K), lambda i: (i, 0)),
        ],
        out_shape=[
            jax.ShapeDtypeStruct((N, NB), jnp.float32),
            jax.ShapeDtypeStruct((N, K), jnp.int32),
        ],
    )(dst_pad, srcT_pad)


def _sc_compact_body(d2v_hbm, csel_hbm, out_hbm, cselv, idxv, buf, sem):
    cc = lax.axis_index("c")
    ss = lax.axis_index("s")
    w = ss * 2 + cc
    row0 = w * RW
    pltpu.sync_copy(csel_hbm.at[pl.ds(row0 * K, RW * K)], cselv)

    def rnd(g, carry):
        r0 = row0 + g * 8
        for t in range(8):
            r = r0 + t
            rc = jnp.minimum(r, N - 1)
            cid = cselv[pl.ds((g * 8 + t) * K, K)]
            idxv[pl.ds(t * K, K)] = rc * C + cid
        pltpu.async_copy(d2v_hbm.at[idxv], buf, sem).wait()
        pltpu.sync_copy(buf, out_hbm.at[pl.ds(r0 * K, 128)])
        return carry

    lax.fori_loop(0, RW // 8, rnd, 0)


@functools.lru_cache(maxsize=1)
def _sc_compact_kernel():
  return pl.kernel(
    _sc_compact_body,
    out_type=jax.ShapeDtypeStruct((NP * K, S), jnp.float32),
    mesh=plsc.VectorSubcoreMesh(core_axis_name="c", subcore_axis_name="s",
                                num_cores=2, num_subcores=16),
    compiler_params=pltpu.CompilerParams(needs_layout_passes=False),
    scratch_types=[
        pltpu.VMEM((RW * K,), jnp.int32),
        pltpu.VMEM((128,), jnp.int32),
        pltpu.VMEM((128, S), jnp.float32),
        pltpu.SemaphoreType.DMA,
    ],
  )


def _tc_c_body(cpt_ref, csel_ref, idx_ref, dsti_ref):
    cpt = cpt_ref[...]
    csel = csel_ref[...]
    iota = lax.broadcasted_iota(jnp.int32, (R, CW), 1)
    iota16 = lax.broadcasted_iota(jnp.int32, (R, K), 1)
    inf = jnp.float32(jnp.inf)
    vals, poss = [], []
    for _ in range(K + 1):
        v = jnp.min(cpt, axis=1, keepdims=True)
        p = jnp.argmin(cpt, axis=1).astype(jnp.int32)[:, None]
        cpt = jnp.where(iota == p, inf, cpt)
        vals.append(v)
        poss.append(p)
    for a, b in _SORT17:
        swap = (vals[b] < vals[a]) | ((vals[b] == vals[a]) & (poss[b] < poss[a]))
        vlo = jnp.where(swap, vals[b], vals[a])
        vhi = jnp.where(swap, vals[a], vals[b])
        plo = jnp.where(swap, poss[b], poss[a])
        phi = jnp.where(swap, poss[a], poss[b])
        vals[a], vals[b] = vlo, vhi
        poss[a], poss[b] = plo, phi
    cols = []
    for n in range(K):
        p = poss[n]
        j = p >> 7
        l = p & 127
        cid = jnp.sum(jnp.where(iota16 == j, csel, 0), axis=1, keepdims=True)
        cols.append(cid * S + l)
    idx_ref[...] = jnp.concatenate(cols, axis=1)
    row0 = pl.program_id(0) * R
    dsti_ref[...] = row0 + lax.broadcasted_iota(jnp.int32, (R, K), 0)


@jax.jit
def _tc_c(cpt, csel):
    return pl.pallas_call(
        _tc_c_body,
        grid=(N // R,),
        in_specs=[
            pl.BlockSpec((R, CW), lambda i: (i, 0)),
            pl.BlockSpec((R, K), lambda i: (i, 0)),
        ],
        out_specs=[
            pl.BlockSpec((R, K), lambda i: (i, 0)),
            pl.BlockSpec((R, K), lambda i: (i, 0)),
        ],
        out_shape=[
            jax.ShapeDtypeStruct((N, K), jnp.int32),
            jax.ShapeDtypeStruct((N, K), jnp.int32),
        ],
    )(cpt, csel)


def _sc_attr_body(sx_hbm, sy_hbm, sz_hbm, dx_hbm, dy_hbm, dz_hbm, idx_hbm,
                  a0_hbm, a1_hbm, a2_hbm, a3_hbm,
                  sxv, syv, szv, dxv, dyv, dzv, idxv, a0, a1, a2, a3):
    cc = lax.axis_index("c")
    ss = lax.axis_index("s")
    w = ss * 2 + cc
    row0 = w * RW
    pltpu.sync_copy(sx_hbm, sxv)
    pltpu.sync_copy(sy_hbm, syv)
    pltpu.sync_copy(sz_hbm, szv)
    pltpu.sync_copy(dx_hbm, dxv)
    pltpu.sync_copy(dy_hbm, dyv)
    pltpu.sync_copy(dz_hbm, dzv)
    pltpu.sync_copy(idx_hbm.at[pl.ds(row0 * K, EW)], idxv)

    def body(r, carry):
        i = row0 + r
        si = idxv[pl.ds(r * K, K)]
        iv = jnp.full((K,), i, jnp.int32)
        dx = plsc.load_gather(dxv, [iv]) - plsc.load_gather(sxv, [si])
        dy = plsc.load_gather(dyv, [iv]) - plsc.load_gather(syv, [si])
        dz = plsc.load_gather(dzv, [iv]) - plsc.load_gather(szv, [si])
        d2 = dx * dx + dy * dy + dz * dz + jnp.float32(1e-12)
        yi = jnp.int32(0x5F3759DF) - lax.shift_right_logical(
            plsc.bitcast(d2, jnp.int32), 1)
        y = plsc.bitcast(yi, jnp.float32)
        h = jnp.float32(0.5) * d2
        for _ in range(3):
            y = y * (jnp.float32(1.5) - h * y * y)
        dist = d2 * y
        e0 = r * K
        a0[pl.ds(e0, K)] = dx
        a1[pl.ds(e0, K)] = dy
        a2[pl.ds(e0, K)] = dz
        a3[pl.ds(e0, K)] = dist
        return carry

    lax.fori_loop(0, RW, body, 0)
    pltpu.sync_copy(a0, a0_hbm.at[pl.ds(row0 * K, EW)])
    pltpu.sync_copy(a1, a1_hbm.at[pl.ds(row0 * K, EW)])
    pltpu.sync_copy(a2, a2_hbm.at[pl.ds(row0 * K, EW)])
    pltpu.sync_copy(a3, a3_hbm.at[pl.ds(row0 * K, EW)])


@functools.lru_cache(maxsize=1)
def _sc_attr_kernel():
  return pl.kernel(
    _sc_attr_body,
    out_type=[jax.ShapeDtypeStruct((EP,), jnp.float32)] * 4,
    mesh=plsc.VectorSubcoreMesh(core_axis_name="c", subcore_axis_name="s",
                                num_cores=2, num_subcores=16),
    compiler_params=pltpu.CompilerParams(needs_layout_passes=False),
    scratch_types=[
        pltpu.VMEM((N,), jnp.float32),
        pltpu.VMEM((N,), jnp.float32),
        pltpu.VMEM((N,), jnp.float32),
        pltpu.VMEM((NP,), jnp.float32),
        pltpu.VMEM((NP,), jnp.float32),
        pltpu.VMEM((NP,), jnp.float32),
        pltpu.VMEM((EW,), jnp.int32),
        pltpu.VMEM((EW,), jnp.float32),
        pltpu.VMEM((EW,), jnp.float32),
        pltpu.VMEM((EW,), jnp.float32),
        pltpu.VMEM((EW,), jnp.float32),
    ],
  )


def kernel(src_coords, dst_coords):
    dst_pad = jnp.pad(dst_coords, ((0, 0), (0, 5)))
    srcT_pad = jnp.pad(src_coords.T, ((0, 5), (0, 0)))
    d2p, csel = _tc_a(dst_pad, srcT_pad)
    d2v = d2p.reshape(N * C, S)
    cselp = jnp.pad(csel, ((0, NP - N), (0, 0))).reshape(-1)
    cpt_rows = _sc_compact_kernel()(d2v, cselp)
    cpt = cpt_rows.reshape(NP, CW)[:N]
    idx, dsti = _tc_c(cpt, csel)
    src_idx = idx.reshape(-1)
    dst_idx = dsti.reshape(-1)
    edge_index = jnp.stack([src_idx, dst_idx], axis=0)
    dpadT = jnp.pad(dst_coords.T, ((0, 0), (0, NP - N)))
    idx_flat = jnp.pad(idx, ((0, NP - N), (0, 0))).reshape(-1)
    a0, a1, a2, a3 = _sc_attr_kernel()(
        src_coords[:, 0], src_coords[:, 1], src_coords[:, 2],
        dpadT[0], dpadT[1], dpadT[2], idx_flat)
    edge_attr = jnp.stack(
        [a0[: N * K], a1[: N * K], a2[: N * K], a3[: N * K]], axis=1)
    return edge_attr, edge_index


# speedup vs baseline: 1.4107x; 1.4107x over previous
import functools

import jax
import jax.numpy as jnp
from jax import lax
from jax.experimental import pallas as pl
from jax.experimental.pallas import tpu as pltpu
from jax.experimental.pallas import tpu_sc as plsc

N = 10000
K = 16
R = 200

NW = 32
RW = 320
NP = NW * RW
EW = RW * K
EP = NP * K


def _tc_topk_body(dst_ref, srcT_ref, idx_ref, dsti_ref):
    dblk = dst_ref[...]
    sT = srcT_ref[...]
    dn2 = jnp.sum(dblk * dblk, axis=1, keepdims=True)
    sn2 = jnp.sum(sT * sT, axis=0, keepdims=True)
    cross = jnp.dot(dblk, sT, preferred_element_type=jnp.float32)
    d2 = (dn2 - 2.0 * cross) + sn2
    iota = jax.lax.broadcasted_iota(jnp.int32, (R, N), 1)
    bigi = jnp.int32(2**30)
    inf = jnp.float32(jnp.inf)
    idx_cols = []
    for _ in range(K):
        m = jnp.min(d2, axis=1, keepdims=True)
        sel = jnp.min(jnp.where(d2 == m, iota, bigi), axis=1, keepdims=True)
        d2 = jnp.where(iota == sel, inf, d2)
        idx_cols.append(sel)
    idx_ref[...] = jnp.concatenate(idx_cols, axis=1)
    row0 = pl.program_id(0) * R
    rows = row0 + jax.lax.broadcasted_iota(jnp.int32, (R, K), 0)
    dsti_ref[...] = rows


@functools.partial(jax.jit, static_argnames=("interpret",))
def _tc_topk(dst_pad, srcT_pad, interpret=False):
    return pl.pallas_call(
        _tc_topk_body,
        grid=(N // R,),
        in_specs=[
            pl.BlockSpec((R, 8), lambda i: (i, 0)),
            pl.BlockSpec((8, N), lambda i: (0, 0)),
        ],
        out_specs=[
            pl.BlockSpec((R, K), lambda i: (i, 0)),
            pl.BlockSpec((R, K), lambda i: (i, 0)),
        ],
        out_shape=[
            jax.ShapeDtypeStruct((N, K), jnp.int32),
            jax.ShapeDtypeStruct((N, K), jnp.int32),
        ],
        interpret=interpret,
    )(dst_pad, srcT_pad)


def _sc_attr_body(sx_hbm, sy_hbm, sz_hbm, dx_hbm, dy_hbm, dz_hbm, idx_hbm,
                  a0_hbm, a1_hbm, a2_hbm, a3_hbm,
                  sxv, syv, szv, dxv, dyv, dzv, idxv, a0, a1, a2, a3):
    c = lax.axis_index("c")
    s = lax.axis_index("s")
    w = s * 2 + c
    row0 = w * RW
    pltpu.sync_copy(sx_hbm, sxv)
    pltpu.sync_copy(sy_hbm, syv)
    pltpu.sync_copy(sz_hbm, szv)
    pltpu.sync_copy(dx_hbm, dxv)
    pltpu.sync_copy(dy_hbm, dyv)
    pltpu.sync_copy(dz_hbm, dzv)
    pltpu.sync_copy(idx_hbm.at[pl.ds(row0 * K, EW)], idxv)

    def body(r, carry):
        i = row0 + r
        si = idxv[pl.ds(r * K, K)]
        iv = jnp.full((K,), i, jnp.int32)
        dx = plsc.load_gather(dxv, [iv]) - plsc.load_gather(sxv, [si])
        dy = plsc.load_gather(dyv, [iv]) - plsc.load_gather(syv, [si])
        dz = plsc.load_gather(dzv, [iv]) - plsc.load_gather(szv, [si])
        d2 = dx * dx + dy * dy + dz * dz + jnp.float32(1e-12)
        yi = jnp.int32(0x5F3759DF) - lax.shift_right_logical(
            plsc.bitcast(d2, jnp.int32), 1)
        y = plsc.bitcast(yi, jnp.float32)
        h = jnp.float32(0.5) * d2
        for _ in range(3):
            y = y * (jnp.float32(1.5) - h * y * y)
        dist = d2 * y
        e0 = r * K
        a0[pl.ds(e0, K)] = dx
        a1[pl.ds(e0, K)] = dy
        a2[pl.ds(e0, K)] = dz
        a3[pl.ds(e0, K)] = dist
        return carry

    lax.fori_loop(0, RW, body, 0)
    pltpu.sync_copy(a0, a0_hbm.at[pl.ds(row0 * K, EW)])
    pltpu.sync_copy(a1, a1_hbm.at[pl.ds(row0 * K, EW)])
    pltpu.sync_copy(a2, a2_hbm.at[pl.ds(row0 * K, EW)])
    pltpu.sync_copy(a3, a3_hbm.at[pl.ds(row0 * K, EW)])


@functools.lru_cache(maxsize=1)
def _sc_attr_kernel():
  return pl.kernel(
    _sc_attr_body,
    out_type=[jax.ShapeDtypeStruct((EP,), jnp.float32)] * 4,
    mesh=plsc.VectorSubcoreMesh(core_axis_name="c", subcore_axis_name="s",
                                num_cores=2, num_subcores=16),
    compiler_params=pltpu.CompilerParams(needs_layout_passes=False),
    scratch_types=[
        pltpu.VMEM((N,), jnp.float32),
        pltpu.VMEM((N,), jnp.float32),
        pltpu.VMEM((N,), jnp.float32),
        pltpu.VMEM((NP,), jnp.float32),
        pltpu.VMEM((NP,), jnp.float32),
        pltpu.VMEM((NP,), jnp.float32),
        pltpu.VMEM((EW,), jnp.int32),
        pltpu.VMEM((EW,), jnp.float32),
        pltpu.VMEM((EW,), jnp.float32),
        pltpu.VMEM((EW,), jnp.float32),
        pltpu.VMEM((EW,), jnp.float32),
    ],
  )


def kernel(src_coords, dst_coords, interpret=False):
    dst_pad = jnp.pad(dst_coords, ((0, 0), (0, 5)))
    srcT_pad = jnp.pad(src_coords.T, ((0, 5), (0, 0)))
    idx, dsti = _tc_topk(dst_pad, srcT_pad, interpret=interpret)
    src_idx = idx.reshape(-1)
    dst_idx = dsti.reshape(-1)
    edge_index = jnp.stack([src_idx, dst_idx], axis=0)
    dpadT = jnp.pad(dst_coords.T, ((0, 0), (0, NP - N)))
    idx_flat = jnp.pad(idx, ((0, NP - N), (0, 0))).reshape(-1)
    a0, a1, a2, a3 = _sc_attr_kernel()(
        src_coords[:, 0], src_coords[:, 1], src_coords[:, 2],
        dpadT[0], dpadT[1], dpadT[2], idx_flat)
    edge_attr = jnp.stack(
        [a0[: N * K], a1[: N * K], a2[: N * K], a3[: N * K]], axis=1)
    return edge_attr, edge_index


# speedup vs baseline: 1.4110x; 1.0002x over previous
import functools

import jax
import jax.numpy as jnp
from jax import lax
from jax.experimental import pallas as pl
from jax.experimental.pallas import tpu as pltpu
from jax.experimental.pallas import tpu_sc as plsc

N = 10000
K = 16
R = 200

NW = 32
RW = 320
NP = NW * RW
EW = RW * K
EP = NP * K


def _tc_topk_body(dst_ref, srcT_ref, idx_ref, dsti_ref):
    dblk = dst_ref[...]
    sT = srcT_ref[...]
    dn2 = jnp.sum(dblk * dblk, axis=1, keepdims=True)
    sn2 = jnp.sum(sT * sT, axis=0, keepdims=True)
    cross = jnp.dot(dblk, sT, preferred_element_type=jnp.float32)
    d2 = (dn2 - 2.0 * cross) + sn2
    iota = jax.lax.broadcasted_iota(jnp.int32, (R, N), 1)
    bigi = jnp.int32(2**30)
    inf = jnp.float32(jnp.inf)
    idx_cols = []
    for _ in range(K):
        m = jnp.min(d2, axis=1, keepdims=True)
        sel = jnp.min(jnp.where(d2 == m, iota, bigi), axis=1, keepdims=True)
        d2 = jnp.where(iota == sel, inf, d2)
        idx_cols.append(sel)
    idx_ref[...] = jnp.concatenate(idx_cols, axis=1)
    row0 = pl.program_id(0) * R
    rows = row0 + jax.lax.broadcasted_iota(jnp.int32, (R, K), 0)
    dsti_ref[...] = rows


@jax.jit
def _tc_topk(dst_pad, srcT_pad):
    return pl.pallas_call(
        _tc_topk_body,
        grid=(N // R,),
        in_specs=[
            pl.BlockSpec((R, 8), lambda i: (i, 0)),
            pl.BlockSpec((8, N), lambda i: (0, 0)),
        ],
        out_specs=[
            pl.BlockSpec((R, K), lambda i: (i, 0)),
            pl.BlockSpec((R, K), lambda i: (i, 0)),
        ],
        out_shape=[
            jax.ShapeDtypeStruct((N, K), jnp.int32),
            jax.ShapeDtypeStruct((N, K), jnp.int32),
        ],
    )(dst_pad, srcT_pad)


def _sc_attr_body(sx_hbm, sy_hbm, sz_hbm, dx_hbm, dy_hbm, dz_hbm, idx_hbm,
                  a0_hbm, a1_hbm, a2_hbm, a3_hbm,
                  sxv, syv, szv, dxv, dyv, dzv, idxv, a0, a1, a2, a3):
    c = lax.axis_index("c")
    s = lax.axis_index("s")
    w = s * 2 + c
    row0 = w * RW
    pltpu.sync_copy(sx_hbm, sxv)
    pltpu.sync_copy(sy_hbm, syv)
    pltpu.sync_copy(sz_hbm, szv)
    pltpu.sync_copy(dx_hbm, dxv)
    pltpu.sync_copy(dy_hbm, dyv)
    pltpu.sync_copy(dz_hbm, dzv)
    pltpu.sync_copy(idx_hbm.at[pl.ds(row0 * K, EW)], idxv)

    def body(r, carry):
        i = row0 + r
        si = idxv[pl.ds(r * K, K)]
        iv = jnp.full((K,), i, jnp.int32)
        dx = plsc.load_gather(dxv, [iv]) - plsc.load_gather(sxv, [si])
        dy = plsc.load_gather(dyv, [iv]) - plsc.load_gather(syv, [si])
        dz = plsc.load_gather(dzv, [iv]) - plsc.load_gather(szv, [si])
        d2 = dx * dx + dy * dy + dz * dz + jnp.float32(1e-12)
        yi = jnp.int32(0x5F3759DF) - lax.shift_right_logical(
            plsc.bitcast(d2, jnp.int32), 1)
        y = plsc.bitcast(yi, jnp.float32)
        h = jnp.float32(0.5) * d2
        for _ in range(3):
            y = y * (jnp.float32(1.5) - h * y * y)
        dist = d2 * y
        e0 = r * K
        a0[pl.ds(e0, K)] = dx
        a1[pl.ds(e0, K)] = dy
        a2[pl.ds(e0, K)] = dz
        a3[pl.ds(e0, K)] = dist
        return carry

    lax.fori_loop(0, RW, body, 0)
    pltpu.sync_copy(a0, a0_hbm.at[pl.ds(row0 * K, EW)])
    pltpu.sync_copy(a1, a1_hbm.at[pl.ds(row0 * K, EW)])
    pltpu.sync_copy(a2, a2_hbm.at[pl.ds(row0 * K, EW)])
    pltpu.sync_copy(a3, a3_hbm.at[pl.ds(row0 * K, EW)])


@functools.lru_cache(maxsize=1)
def _sc_attr_kernel():
  return pl.kernel(
    _sc_attr_body,
    out_type=[jax.ShapeDtypeStruct((EP,), jnp.float32)] * 4,
    mesh=plsc.VectorSubcoreMesh(core_axis_name="c", subcore_axis_name="s",
                                num_cores=2, num_subcores=16),
    compiler_params=pltpu.CompilerParams(needs_layout_passes=False),
    scratch_types=[
        pltpu.VMEM((N,), jnp.float32),
        pltpu.VMEM((N,), jnp.float32),
        pltpu.VMEM((N,), jnp.float32),
        pltpu.VMEM((NP,), jnp.float32),
        pltpu.VMEM((NP,), jnp.float32),
        pltpu.VMEM((NP,), jnp.float32),
        pltpu.VMEM((EW,), jnp.int32),
        pltpu.VMEM((EW,), jnp.float32),
        pltpu.VMEM((EW,), jnp.float32),
        pltpu.VMEM((EW,), jnp.float32),
        pltpu.VMEM((EW,), jnp.float32),
    ],
  )


def kernel(src_coords, dst_coords):
    dst_pad = jnp.pad(dst_coords, ((0, 0), (0, 5)))
    srcT_pad = jnp.pad(src_coords.T, ((0, 5), (0, 0)))
    idx, dsti = _tc_topk(dst_pad, srcT_pad)
    src_idx = idx.reshape(-1)
    dst_idx = dsti.reshape(-1)
    edge_index = jnp.stack([src_idx, dst_idx], axis=0)
    dpadT = jnp.pad(dst_coords.T, ((0, 0), (0, NP - N)))
    idx_flat = jnp.pad(idx, ((0, NP - N), (0, 0))).reshape(-1)
    a0, a1, a2, a3 = _sc_attr_kernel()(
        src_coords[:, 0], src_coords[:, 1], src_coords[:, 2],
        dpadT[0], dpadT[1], dpadT[2], idx_flat)
    edge_attr = jnp.stack(
        [a0[: N * K], a1[: N * K], a2[: N * K], a3[: N * K]], axis=1)
    return edge_attr, edge_index
